# baseline (speedup 1.0000x reference)
import jax
import jax.numpy as jnp
from jax import lax
from jax.experimental import pallas as pl
from jax.experimental.pallas import tpu as pltpu
from jax.experimental.pallas import tpu_sc as plsc

NC = 2
NS = 16
L = 16

B = 16
N = 32768
NPOINT = 512
HALF = N // 2
CHUNKS = HALF // L
U = 8

REPUB_TRIPS = 8
POLL_TRIPS = 64


def _lane(vec, lane):
    idx = jnp.full((L,), lane, jnp.int32)
    dnums = lax.GatherDimensionNumbers(
        offset_dims=(), collapsed_slice_dims=(0,), start_index_map=(0,))
    return lax.gather(vec, idx[:, None], dnums, (1,),
                      mode=lax.GatherScatterMode.PROMISE_IN_BOUNDS)[0]


def _fps_body(xyz_hbm, out_hbm, x_ref, y_ref, z_ref, d_ref, idx_ref,
              rec_ref, prec_ref, shared_ref):
    c = lax.axis_index("c")
    s = lax.axis_index("s")
    b = c * (NS // 2) + s // 2
    h = s % 2

    base = (b * 3) * N
    pltpu.sync_copy(xyz_hbm.at[pl.ds(base, N)], x_ref)
    pltpu.sync_copy(xyz_hbm.at[pl.ds(base + N, N)], y_ref)
    pltpu.sync_copy(xyz_hbm.at[pl.ds(base + 2 * N, N)], z_ref)

    iot = lax.iota(jnp.int32, L)

    rec_ref[...] = jnp.full((L,), -2.0, jnp.float32)
    pltpu.sync_copy(rec_ref, shared_ref.at[NS + s, pl.ds(0, L)])

    big = jnp.full((L,), 1e10, jnp.float32)

    def init_body(k, carry):
        d_ref[pl.ds(k * L, L)] = big
        return carry

    lax.fori_loop(0, CHUNKS, init_body, 0)

    def exchange(tag, bv, bj):
        rec = jnp.where(iot == 0, bv,
              jnp.where(iot == 1, bj.astype(jnp.float32), tag))
        rec_ref[...] = rec
        pltpu.sync_copy(rec_ref, shared_ref.at[NS + s, pl.ds(0, L)])
        pltpu.sync_copy(shared_ref.at[NS + s, pl.ds(0, L)], prec_ref)

        def repub(t, q):
            bad = q[2] != tag

            @pl.when(bad)
            def _():
                pltpu.sync_copy(rec_ref, shared_ref.at[NS + s, pl.ds(0, L)])
                pltpu.sync_copy(shared_ref.at[NS + s, pl.ds(0, L)], prec_ref)

            return jnp.where(bad, prec_ref[...], q)

        lax.fori_loop(0, REPUB_TRIPS, repub, prec_ref[...])

        pltpu.sync_copy(shared_ref.at[NS + (s ^ 1), pl.ds(0, L)], prec_ref)

        def poll(t, p):
            bad = p[2] != tag

            @pl.when(bad)
            def _():
                pltpu.sync_copy(shared_ref.at[NS + (s ^ 1), pl.ds(0, L)], prec_ref)

            return jnp.where(bad, prec_ref[...], p)

        p = lax.fori_loop(0, POLL_TRIPS, poll, prec_ref[...])
        pv = p[0]
        pj = p[1].astype(jnp.int32)
        take = (pv > bv) | ((pv == bv) & (pj < bj))
        return jnp.where(take, pv, bv), jnp.where(take, pj, bj)

    x0 = x_ref[pl.ds(0, L)][0]
    y0 = y_ref[pl.ds(0, L)][0]
    z0 = z_ref[pl.ds(0, L)][0]

    kv0 = iot + h * HALF
    neg = jnp.full((L,), -1.0, jnp.float32)
    gbase = h * HALF

    def iter_body(i, carry):
        cx, cy, cz, gi, idxacc = carry
        idxacc = jnp.where(iot == i % L, gi, idxacc)

        @pl.when(i % L == L - 1)
        def _():
            idx_ref[pl.ds((i // L) * L, L)] = idxacc

        def chunk_body(k, ch):
            bests, bidxs, kv = ch
            lo = k * (L * U)
            newb = []
            newi = []
            for j in range(U):
                dsl = pl.ds(lo + j * L, L)
                gsl = pl.ds(gbase + lo + j * L, L)
                dx = x_ref[gsl] - cx
                dy = y_ref[gsl] - cy
                dz = z_ref[gsl] - cz
                dist = dx * dx + dy * dy + dz * dz
                nd = jnp.minimum(d_ref[dsl], dist)
                d_ref[dsl] = nd
                kvj = kv + j * L
                m = nd > bests[j]
                newb.append(jnp.where(m, nd, bests[j]))
                newi.append(jnp.where(m, kvj, bidxs[j]))
            return tuple(newb), tuple(newi), kv + L * U

        bests, bidxs, _ = lax.fori_loop(
            0, CHUNKS // U, chunk_body,
            ((neg,) * U, (kv0,) * U, kv0))

        def comb(a, bcc):
            b1, i1 = a
            b2, i2 = bcc
            m = (b2 > b1) | ((b2 == b1) & (i2 < i1))
            return jnp.where(m, b2, b1), jnp.where(m, i2, i1)

        acc = list(zip(bests, bidxs))
        while len(acc) > 1:
            acc = [comb(acc[t], acc[t + 1]) for t in range(0, len(acc), 2)]
        best, bidx = acc[0]

        bv = jnp.max(best)
        bj = jnp.min(jnp.where(best == bv, bidx, jnp.int32(N)))
        wv, wj = exchange((i + 1).astype(jnp.float32), bv, bj)

        wchunk = (wj // L) * L
        wlane = wj - wchunk
        ncx = _lane(x_ref[pl.ds(wchunk, L)], wlane)
        ncy = _lane(y_ref[pl.ds(wchunk, L)], wlane)
        ncz = _lane(z_ref[pl.ds(wchunk, L)], wlane)
        return ncx, ncy, ncz, wj, idxacc

    carry0 = (x0, y0, z0, jnp.int32(0), jnp.zeros((L,), jnp.int32))
    lax.fori_loop(0, NPOINT, iter_body, carry0)

    @pl.when(h == 0)
    def _():
        pltpu.sync_copy(idx_ref, out_hbm.at[pl.ds(b * NPOINT, NPOINT)])


_fps_call = pl.kernel(
    _fps_body,
    out_type=jax.ShapeDtypeStruct((B * NPOINT,), jnp.int32),
    mesh=plsc.VectorSubcoreMesh(core_axis_name="c", subcore_axis_name="s"),
    compiler_params=pltpu.CompilerParams(needs_layout_passes=False),
    scratch_types=[
        pltpu.VMEM((N,), jnp.float32),
        pltpu.VMEM((N,), jnp.float32),
        pltpu.VMEM((N,), jnp.float32),
        pltpu.VMEM((HALF,), jnp.float32),
        pltpu.VMEM((NPOINT,), jnp.int32),
        pltpu.VMEM((L,), jnp.float32),
        pltpu.VMEM((L,), jnp.float32),
        pltpu.VMEM_SHARED((2 * NS, 2 * L), jnp.float32),
    ],
)


@jax.jit
def _fps(xyz):
    xt = jnp.transpose(xyz, (0, 2, 1)).reshape(-1)
    return _fps_call(xt).reshape(B, NPOINT)


def kernel(xyz, points):
    del points
    return _fps(xyz)


# speedup vs baseline: 2.6772x; 2.6772x over previous
import jax
import jax.numpy as jnp
from jax import lax
from jax.experimental import pallas as pl
from jax.experimental.pallas import tpu as pltpu
from jax.experimental.pallas import tpu_sc as plsc

NC = 2
NS = 16
L = 16

B = 16
N = 32768
NPOINT = 512
HALF = N // 2
CHUNKS = HALF // L
U = 8

REPUB_TRIPS = 8
POLL_TRIPS = 64


def _lane(vec, lane):
    idx = jnp.full((L,), lane, jnp.int32)
    dnums = lax.GatherDimensionNumbers(
        offset_dims=(), collapsed_slice_dims=(0,), start_index_map=(0,))
    return lax.gather(vec, idx[:, None], dnums, (1,),
                      mode=lax.GatherScatterMode.PROMISE_IN_BOUNDS)[0]


def _fps_body(xyz_hbm, out_hbm, x_ref, y_ref, z_ref, d_ref, idx_ref,
              rec_ref, prec_ref, shared_ref):
    c = lax.axis_index("c")
    s = lax.axis_index("s")
    b = c * (NS // 2) + s // 2
    h = s % 2

    base = (b * 3) * N
    pltpu.sync_copy(xyz_hbm.at[pl.ds(base, N)], x_ref)
    pltpu.sync_copy(xyz_hbm.at[pl.ds(base + N, N)], y_ref)
    pltpu.sync_copy(xyz_hbm.at[pl.ds(base + 2 * N, N)], z_ref)

    iot = lax.iota(jnp.int32, L)

    rec_ref[...] = jnp.full((L,), -2.0, jnp.float32)
    pltpu.sync_copy(rec_ref, shared_ref.at[NS + s, pl.ds(0, L)])

    big = jnp.full((L,), 1e10, jnp.float32)

    def init_body(k, carry):
        d_ref[pl.ds(k * L, L)] = big
        return carry

    lax.fori_loop(0, CHUNKS, init_body, 0)

    def exchange(tag, bv, bj):
        rec = jnp.where(iot == 0, bv,
              jnp.where(iot == 1, bj.astype(jnp.float32), tag))
        rec_ref[...] = rec
        pltpu.sync_copy(rec_ref, shared_ref.at[NS + s, pl.ds(0, L)])
        pltpu.sync_copy(shared_ref.at[NS + s, pl.ds(0, L)], prec_ref)

        def repub(t, q):
            bad = q[2] != tag

            @pl.when(bad)
            def _():
                pltpu.sync_copy(rec_ref, shared_ref.at[NS + s, pl.ds(0, L)])
                pltpu.sync_copy(shared_ref.at[NS + s, pl.ds(0, L)], prec_ref)

            return jnp.where(bad, prec_ref[...], q)

        lax.fori_loop(0, REPUB_TRIPS, repub, prec_ref[...])

        pltpu.sync_copy(shared_ref.at[NS + (s ^ 1), pl.ds(0, L)], prec_ref)

        def poll(t, p):
            bad = p[2] != tag

            @pl.when(bad)
            def _():
                pltpu.sync_copy(shared_ref.at[NS + (s ^ 1), pl.ds(0, L)], prec_ref)

            return jnp.where(bad, prec_ref[...], p)

        p = lax.fori_loop(0, POLL_TRIPS, poll, prec_ref[...])
        pv = p[0]
        pj = p[1].astype(jnp.int32)
        take = (pv > bv) | ((pv == bv) & (pj < bj))
        return jnp.where(take, pv, bv), jnp.where(take, pj, bj)

    x0 = x_ref[pl.ds(0, L)][0]
    y0 = y_ref[pl.ds(0, L)][0]
    z0 = z_ref[pl.ds(0, L)][0]

    kv0 = iot + h * HALF
    neg = jnp.full((L,), -1.0, jnp.float32)
    gbase = h * HALF

    def iter_body(i, carry):
        cx, cy, cz, gi, idxacc = carry
        idxacc = jnp.where(iot == i % L, gi, idxacc)

        @pl.when(i % L == L - 1)
        def _():
            idx_ref[pl.ds((i // L) * L, L)] = idxacc

        def chunk_body(k, ch):
            bests, bidxs, kv = ch
            lo = k * (L * U)
            newb = []
            newi = []
            for j in range(U):
                dsl = pl.ds(lo + j * L, L)
                gsl = pl.ds(gbase + lo + j * L, L)
                dx = x_ref[gsl] - cx
                dy = y_ref[gsl] - cy
                dz = z_ref[gsl] - cz
                dist = dx * dx + dy * dy + dz * dz
                nd = jnp.minimum(d_ref[dsl], dist)
                d_ref[dsl] = nd
                kvj = kv + j * L
                m = nd > bests[j]
                newb.append(jnp.where(m, nd, bests[j]))
                newi.append(jnp.where(m, kvj, bidxs[j]))
            return tuple(newb), tuple(newi), kv + L * U

        bests, bidxs, _ = plsc.parallel_loop(
            0, CHUNKS // U, 1, unroll=1,
            carry=((neg,) * U, (kv0,) * U, kv0))(
                lambda k, ch: chunk_body(k, ch))

        def comb(a, bcc):
            b1, i1 = a
            b2, i2 = bcc
            m = (b2 > b1) | ((b2 == b1) & (i2 < i1))
            return jnp.where(m, b2, b1), jnp.where(m, i2, i1)

        acc = list(zip(bests, bidxs))
        while len(acc) > 1:
            acc = [comb(acc[t], acc[t + 1]) for t in range(0, len(acc), 2)]
        best, bidx = acc[0]

        bv = jnp.max(best)
        bj = jnp.min(jnp.where(best == bv, bidx, jnp.int32(N)))
        wv, wj = exchange((i + 1).astype(jnp.float32), bv, bj)

        wchunk = (wj // L) * L
        wlane = wj - wchunk
        ncx = _lane(x_ref[pl.ds(wchunk, L)], wlane)
        ncy = _lane(y_ref[pl.ds(wchunk, L)], wlane)
        ncz = _lane(z_ref[pl.ds(wchunk, L)], wlane)
        return ncx, ncy, ncz, wj, idxacc

    carry0 = (x0, y0, z0, jnp.int32(0), jnp.zeros((L,), jnp.int32))
    lax.fori_loop(0, NPOINT, iter_body, carry0)

    @pl.when(h == 0)
    def _():
        pltpu.sync_copy(idx_ref, out_hbm.at[pl.ds(b * NPOINT, NPOINT)])


_fps_call = pl.kernel(
    _fps_body,
    out_type=jax.ShapeDtypeStruct((B * NPOINT,), jnp.int32),
    mesh=plsc.VectorSubcoreMesh(core_axis_name="c", subcore_axis_name="s"),
    compiler_params=pltpu.CompilerParams(needs_layout_passes=False),
    scratch_types=[
        pltpu.VMEM((N,), jnp.float32),
        pltpu.VMEM((N,), jnp.float32),
        pltpu.VMEM((N,), jnp.float32),
        pltpu.VMEM((HALF,), jnp.float32),
        pltpu.VMEM((NPOINT,), jnp.int32),
        pltpu.VMEM((L,), jnp.float32),
        pltpu.VMEM((L,), jnp.float32),
        pltpu.VMEM_SHARED((2 * NS, 2 * L), jnp.float32),
    ],
)


@jax.jit
def _fps(xyz):
    xt = jnp.transpose(xyz, (0, 2, 1)).reshape(-1)
    return _fps_call(xt).reshape(B, NPOINT)


def kernel(xyz, points):
    del points
    return _fps(xyz)


# speedup vs baseline: 4.0160x; 1.5001x over previous
import jax
import jax.numpy as jnp
from jax import lax
from jax.experimental import pallas as pl
from jax.experimental.pallas import tpu as pltpu
from jax.experimental.pallas import tpu_sc as plsc

NC = 2
NS = 16
L = 16

B = 16
N = 32768
NPOINT = 512
HALF = N // 2
CHUNKS = HALF // L
U = 8

REPUB_TRIPS = 8
POLL_TRIPS = 64


def _lane(vec, lane):
    idx = jnp.full((L,), lane, jnp.int32)
    dnums = lax.GatherDimensionNumbers(
        offset_dims=(), collapsed_slice_dims=(0,), start_index_map=(0,))
    return lax.gather(vec, idx[:, None], dnums, (1,),
                      mode=lax.GatherScatterMode.PROMISE_IN_BOUNDS)[0]


def _fps_body(xyz_hbm, out_hbm, x_ref, y_ref, z_ref, d_ref, idx_ref,
              rec_ref, prec_ref, shared_ref):
    c = lax.axis_index("c")
    s = lax.axis_index("s")
    b = c * (NS // 2) + s // 2
    h = s % 2

    base = (b * 3) * N
    pltpu.sync_copy(xyz_hbm.at[pl.ds(base, N)], x_ref)
    pltpu.sync_copy(xyz_hbm.at[pl.ds(base + N, N)], y_ref)
    pltpu.sync_copy(xyz_hbm.at[pl.ds(base + 2 * N, N)], z_ref)

    iot = lax.iota(jnp.int32, L)

    rec_ref[...] = jnp.full((L,), -2.0, jnp.float32)
    pltpu.sync_copy(rec_ref, shared_ref.at[NS + s, pl.ds(0, L)])

    big = jnp.full((L,), 1e10, jnp.float32)

    def init_body(k, carry):
        d_ref[pl.ds(k * L, L)] = big
        return carry

    lax.fori_loop(0, CHUNKS, init_body, 0)

    def exchange(tag, bv, bj):
        rec = jnp.where(iot == 0, bv,
              jnp.where(iot == 1, bj.astype(jnp.float32), tag))
        rec_ref[...] = rec
        pltpu.sync_copy(rec_ref, shared_ref.at[NS + s, pl.ds(0, L)])
        pltpu.sync_copy(shared_ref.at[NS + (s ^ 1), pl.ds(0, L)], prec_ref)

        def spin_cond(p):
            return p[2] != tag

        def spin_body(p):
            pltpu.sync_copy(shared_ref.at[NS + (s ^ 1), pl.ds(0, L)],
                            prec_ref)
            return prec_ref[...]

        p = lax.while_loop(spin_cond, spin_body, prec_ref[...])
        pv = p[0]
        pj = p[1].astype(jnp.int32)
        take = (pv > bv) | ((pv == bv) & (pj < bj))
        return jnp.where(take, pv, bv), jnp.where(take, pj, bj)

    x0 = x_ref[pl.ds(0, L)][0]
    y0 = y_ref[pl.ds(0, L)][0]
    z0 = z_ref[pl.ds(0, L)][0]

    kv0 = iot + h * HALF
    neg = jnp.full((L,), -1.0, jnp.float32)
    gbase = h * HALF

    def iter_body(i, carry):
        cx, cy, cz, gi, idxacc = carry
        idxacc = jnp.where(iot == i % L, gi, idxacc)

        @pl.when(i % L == L - 1)
        def _():
            idx_ref[pl.ds((i // L) * L, L)] = idxacc

        def chunk_body(k, ch):
            bests, bidxs, kv = ch
            lo = k * (L * U)
            newb = []
            newi = []
            for j in range(U):
                dsl = pl.ds(lo + j * L, L)
                gsl = pl.ds(gbase + lo + j * L, L)
                dx = x_ref[gsl] - cx
                dy = y_ref[gsl] - cy
                dz = z_ref[gsl] - cz
                dist = dx * dx + dy * dy + dz * dz
                nd = jnp.minimum(d_ref[dsl], dist)
                d_ref[dsl] = nd
                kvj = kv + j * L
                m = nd > bests[j]
                newb.append(jnp.where(m, nd, bests[j]))
                newi.append(jnp.where(m, kvj, bidxs[j]))
            return tuple(newb), tuple(newi), kv + L * U

        bests, bidxs, _ = plsc.parallel_loop(
            0, CHUNKS // U, 1, unroll=1,
            carry=((neg,) * U, (kv0,) * U, kv0))(
                lambda k, ch: chunk_body(k, ch))

        def comb(a, bcc):
            b1, i1 = a
            b2, i2 = bcc
            m = (b2 > b1) | ((b2 == b1) & (i2 < i1))
            return jnp.where(m, b2, b1), jnp.where(m, i2, i1)

        acc = list(zip(bests, bidxs))
        while len(acc) > 1:
            acc = [comb(acc[t], acc[t + 1]) for t in range(0, len(acc), 2)]
        best, bidx = acc[0]

        bv = jnp.max(best)
        bj = jnp.min(jnp.where(best == bv, bidx, jnp.int32(N)))
        wv, wj = exchange((i + 1).astype(jnp.float32), bv, bj)

        wchunk = (wj // L) * L
        wlane = wj - wchunk
        ncx = _lane(x_ref[pl.ds(wchunk, L)], wlane)
        ncy = _lane(y_ref[pl.ds(wchunk, L)], wlane)
        ncz = _lane(z_ref[pl.ds(wchunk, L)], wlane)
        return ncx, ncy, ncz, wj, idxacc

    carry0 = (x0, y0, z0, jnp.int32(0), jnp.zeros((L,), jnp.int32))
    lax.fori_loop(0, NPOINT, iter_body, carry0)

    @pl.when(h == 0)
    def _():
        pltpu.sync_copy(idx_ref, out_hbm.at[pl.ds(b * NPOINT, NPOINT)])


_fps_call = pl.kernel(
    _fps_body,
    out_type=jax.ShapeDtypeStruct((B * NPOINT,), jnp.int32),
    mesh=plsc.VectorSubcoreMesh(core_axis_name="c", subcore_axis_name="s"),
    compiler_params=pltpu.CompilerParams(needs_layout_passes=False),
    scratch_types=[
        pltpu.VMEM((N,), jnp.float32),
        pltpu.VMEM((N,), jnp.float32),
        pltpu.VMEM((N,), jnp.float32),
        pltpu.VMEM((HALF,), jnp.float32),
        pltpu.VMEM((NPOINT,), jnp.int32),
        pltpu.VMEM((L,), jnp.float32),
        pltpu.VMEM((L,), jnp.float32),
        pltpu.VMEM_SHARED((2 * NS, 2 * L), jnp.float32),
    ],
)


@jax.jit
def _fps(xyz):
    xt = jnp.transpose(xyz, (0, 2, 1)).reshape(-1)
    return _fps_call(xt).reshape(B, NPOINT)


def kernel(xyz, points):
    del points
    return _fps(xyz)
